# Initial kernel scaffold; baseline (speedup 1.0000x reference)
import jax
import jax.numpy as jnp
from jax.experimental import pallas as pl


def kernel(x, g1, g2, W1, b1, W2, b2, Wout, bout):
    raise NotImplementedError("write your pallas kernel here")


# speedup vs baseline: 3.6455x; 3.6455x over previous
import functools

import jax
import jax.numpy as jnp
from jax import lax
from jax.experimental import pallas as pl
from jax.experimental.pallas import tpu as pltpu
from jax.experimental.pallas import tpu_sc as plsc

N = 10000
E = 320000
D = 128
H = 128
C = 40

NC = 2
NS = 16
LANE = 128

NPAD = 10240
ROWS_PER_TILE = NPAD // NS
CHK = 32
ER = 160
EPAD = NS * ER * LANE
SINK = N + 16

@functools.cache
def _mesh():
    return plsc.VectorSubcoreMesh(
        core_axis_name="c", subcore_axis_name="s",
        num_cores=NC, num_subcores=NS)


def _zero_vec_ref(ref, n):
    def body(i, _):
        ref[pl.ds(i * 16, 16)] = jnp.zeros((16,), jnp.float32)
        return _
    lax.fori_loop(0, n // 16, body, None)


def _zero_rows_ref(ref):
    def body(k, _):
        i = k // (D // 16)
        l = k % (D // 16)
        ref[i, pl.ds(l * 16, 16)] = jnp.zeros((16,), jnp.float32)
        return _
    lax.fori_loop(0, LANE * (D // 16), body, None)


@functools.cache
def _deg_kernel():
    return pl.kernel(
        _deg,
        out_type=[jax.ShapeDtypeStruct((NPAD,), jnp.float32)] * 4,
        mesh=_mesh(),
        scratch_types=[
            pltpu.VMEM((ER, LANE), jnp.int32),
            pltpu.VMEM((ER, LANE), jnp.int32),
            pltpu.VMEM((LANE,), jnp.float32),
            pltpu.VMEM((ROWS_PER_TILE,), jnp.float32),
            pltpu.VMEM_SHARED((NPAD,), jnp.float32),
            pltpu.VMEM_SHARED((NPAD,), jnp.float32),
        ],
    )


def _deg(src1, dst1, src2, dst2, do1, di1, do2, di2,
         sidx_v, didx_v, ones_v, stage_v, acc_s, acc_d):
    cid = lax.axis_index("c")
    sid = lax.axis_index("s")

    def setup(i, _):
        ones_v[pl.ds(i * 16, 16)] = jnp.ones((16,), jnp.float32)
        return _
    lax.fori_loop(0, LANE // 16, setup, None)
    _zero_vec_ref(stage_v, ROWS_PER_TILE)
    base = sid * ROWS_PER_TILE
    pltpu.sync_copy(stage_v, acc_s.at[pl.ds(base, ROWS_PER_TILE)])
    pltpu.sync_copy(stage_v, acc_d.at[pl.ds(base, ROWS_PER_TILE)])
    plsc.subcore_barrier()

    def count(src3, dst3, out_s, out_d):
        pltpu.sync_copy(src3.at[sid], sidx_v)
        pltpu.sync_copy(dst3.at[sid], didx_v)

        def body(j, _):
            pltpu.sync_copy(ones_v, acc_s.at[sidx_v.at[j]], add=True)
            pltpu.sync_copy(ones_v, acc_d.at[didx_v.at[j]], add=True)
            return _
        lax.fori_loop(0, ER, body, None)
        plsc.subcore_barrier()
        pltpu.sync_copy(acc_s.at[pl.ds(base, ROWS_PER_TILE)], stage_v)
        pltpu.sync_copy(stage_v, out_s.at[pl.ds(base, ROWS_PER_TILE)])
        pltpu.sync_copy(acc_d.at[pl.ds(base, ROWS_PER_TILE)], stage_v)
        pltpu.sync_copy(stage_v, out_d.at[pl.ds(base, ROWS_PER_TILE)])

    @pl.when(cid == 0)
    def _():
        count(src1, dst1, do1, di1)

    @pl.when(cid == 1)
    def _():
        count(src2, dst2, do2, di2)


@functools.cache
def _agg_kernel():
    return pl.kernel(
        _agg,
        out_type=[jax.ShapeDtypeStruct((NPAD, D), jnp.float32)] * 2,
        mesh=_mesh(),
        scratch_types=[
            pltpu.VMEM((CHK, LANE), jnp.int32),
            pltpu.VMEM((CHK, LANE), jnp.int32),
            pltpu.VMEM((LANE, D), jnp.float32),
            pltpu.VMEM_SHARED((NPAD, D), jnp.float32),
        ],
    )


def _agg(x1, x2, src1, dst1, src2, dst2, out1, out2,
         sidx_v, didx_v, rows_v, acc):
    cid = lax.axis_index("c")
    sid = lax.axis_index("s")

    _zero_rows_ref(rows_v)
    base = sid * ROWS_PER_TILE
    for chunk in range(ROWS_PER_TILE // LANE):
        pltpu.sync_copy(rows_v, acc.at[pl.ds(base + chunk * LANE, LANE)])
    plsc.subcore_barrier()

    def run(x_ref, src3, dst3, out_ref):
        def chunk_body(c, _):
            pltpu.sync_copy(src3.at[sid, pl.ds(c * CHK, CHK)], sidx_v)
            pltpu.sync_copy(dst3.at[sid, pl.ds(c * CHK, CHK)], didx_v)

            def body(j, _):
                pltpu.sync_copy(x_ref.at[sidx_v.at[j]], rows_v)
                pltpu.sync_copy(rows_v, acc.at[didx_v.at[j]], add=True)
                return _
            lax.fori_loop(0, CHK, body, None)
            return _
        lax.fori_loop(0, ER // CHK, chunk_body, None)
        plsc.subcore_barrier()
        for chunk in range(ROWS_PER_TILE // LANE):
            o = base + chunk * LANE
            pltpu.sync_copy(acc.at[pl.ds(o, LANE)], rows_v)
            pltpu.sync_copy(rows_v, out_ref.at[pl.ds(o, LANE)])

    @pl.when(cid == 0)
    def _():
        run(x1, src1, dst1, out1)

    @pl.when(cid == 1)
    def _():
        run(x2, src2, dst2, out2)


def _scale_body(x_ref, d1_ref, d2_ref, o1_ref, o2_ref):
    x = x_ref[...]
    o1_ref[...] = x * lax.rsqrt(jnp.maximum(d1_ref[...], 1.0))
    o2_ref[...] = x * lax.rsqrt(jnp.maximum(d2_ref[...], 1.0))


def _scale(x_pad, do1, do2):
    blk = 1024
    grid = NPAD // blk
    return pl.pallas_call(
        _scale_body,
        grid=(grid,),
        in_specs=[
            pl.BlockSpec((blk, D), lambda i: (i, 0)),
            pl.BlockSpec((blk, 1), lambda i: (i, 0)),
            pl.BlockSpec((blk, 1), lambda i: (i, 0)),
        ],
        out_specs=[
            pl.BlockSpec((blk, D), lambda i: (i, 0)),
            pl.BlockSpec((blk, D), lambda i: (i, 0)),
        ],
        out_shape=[jax.ShapeDtypeStruct((NPAD, D), jnp.float32)] * 2,
    )(x_pad, do1, do2)


def _head_body(a1_ref, a2_ref, d1_ref, d2_ref, w1_ref, b1_ref, w2_ref,
               b2_ref, wo1_ref, wo2_ref, bo_ref, o_ref):
    n1 = lax.rsqrt(jnp.maximum(d1_ref[...], 1.0))
    n2 = lax.rsqrt(jnp.maximum(d2_ref[...], 1.0))
    h1 = jnp.dot(a1_ref[...] * n1, w1_ref[...],
                 preferred_element_type=jnp.float32) + b1_ref[...]
    h2 = jnp.dot(a2_ref[...] * n2, w2_ref[...],
                 preferred_element_type=jnp.float32) + b2_ref[...]
    h1 = jnp.maximum(h1, 0.0)
    h2 = jnp.maximum(h2, 0.0)
    l = (jnp.dot(h1, wo1_ref[...], preferred_element_type=jnp.float32)
         + jnp.dot(h2, wo2_ref[...], preferred_element_type=jnp.float32)
         + bo_ref[...])
    m = jnp.max(l, axis=1, keepdims=True)
    e = jnp.exp(l - m)
    o_ref[...] = l - m - jnp.log(jnp.sum(e, axis=1, keepdims=True))


def _head(agg1, agg2, di1, di2, W1, b1, W2, b2, Wout, bout):
    blk = 1024
    grid = NPAD // blk
    full = lambda shape: pl.BlockSpec(shape, lambda i: tuple(0 for _ in shape))
    return pl.pallas_call(
        _head_body,
        grid=(grid,),
        in_specs=[
            pl.BlockSpec((blk, D), lambda i: (i, 0)),
            pl.BlockSpec((blk, D), lambda i: (i, 0)),
            pl.BlockSpec((blk, 1), lambda i: (i, 0)),
            pl.BlockSpec((blk, 1), lambda i: (i, 0)),
            full((D, H)),
            full((1, H)),
            full((D, H)),
            full((1, H)),
            full((H, C)),
            full((H, C)),
            full((1, C)),
        ],
        out_specs=pl.BlockSpec((blk, C), lambda i: (i, 0)),
        out_shape=jax.ShapeDtypeStruct((NPAD, C), jnp.float32),
    )(agg1, agg2, di1, di2, W1, b1.reshape(1, H), W2, b2.reshape(1, H),
      Wout[:H], Wout[H:], bout.reshape(1, C))


def _pad_edges(g):
    pad = EPAD - E
    src = jnp.concatenate([g[0], jnp.full((pad,), SINK, jnp.int32)])
    dst = jnp.concatenate([g[1], jnp.full((pad,), SINK, jnp.int32)])
    return src.reshape(NS, ER, LANE), dst.reshape(NS, ER, LANE)


def kernel(x, g1, g2, W1, b1, W2, b2, Wout, bout):
    src1, dst1 = _pad_edges(g1)
    src2, dst2 = _pad_edges(g2)
    x_pad = jnp.pad(x, ((0, NPAD - N), (0, 0)))

    do1, di1, do2, di2 = _deg_kernel()(src1, dst1, src2, dst2)
    do1 = do1.reshape(NPAD, 1)
    di1 = di1.reshape(NPAD, 1)
    do2 = do2.reshape(NPAD, 1)
    di2 = di2.reshape(NPAD, 1)

    x1p, x2p = _scale(x_pad, do1, do2)
    agg1, agg2 = _agg_kernel()(x1p, x2p, src1, dst1, src2, dst2)
    out = _head(agg1, agg2, di1, di2, W1, b1, W2, b2, Wout, bout)
    return out[:N]


# speedup vs baseline: 4.2664x; 1.1703x over previous
import functools

import jax
import jax.numpy as jnp
from jax import lax
from jax.experimental import pallas as pl
from jax.experimental.pallas import tpu as pltpu
from jax.experimental.pallas import tpu_sc as plsc

N = 10000
E = 320000
D = 128
H = 128
C = 40

NC = 2
NS = 16
LANE = 128

NPAD = 10240
ROWS_PER_TILE = NPAD // NS
CHK = 32
ER = 160
EPAD = NS * ER * LANE
SINK = N + 16

@functools.cache
def _mesh():
    return plsc.VectorSubcoreMesh(
        core_axis_name="c", subcore_axis_name="s",
        num_cores=NC, num_subcores=NS)


def _zero_vec_ref(ref, n):
    def body(i, _):
        ref[pl.ds(i * 16, 16)] = jnp.zeros((16,), jnp.float32)
        return _
    lax.fori_loop(0, n // 16, body, None)


def _zero_rows_ref(ref):
    def body(k, _):
        i = k // (D // 16)
        l = k % (D // 16)
        ref[i, pl.ds(l * 16, 16)] = jnp.zeros((16,), jnp.float32)
        return _
    lax.fori_loop(0, LANE * (D // 16), body, None)


@functools.cache
def _deg_kernel():
    return pl.kernel(
        _deg,
        out_type=[jax.ShapeDtypeStruct((NPAD,), jnp.float32)] * 4,
        mesh=_mesh(),
        scratch_types=[
            pltpu.VMEM((ER, LANE), jnp.int32),
            pltpu.VMEM((ER, LANE), jnp.int32),
            pltpu.VMEM((LANE,), jnp.float32),
            pltpu.VMEM((ROWS_PER_TILE,), jnp.float32),
            pltpu.VMEM_SHARED((NPAD,), jnp.float32),
            pltpu.VMEM_SHARED((NPAD,), jnp.float32),
        ],
    )


def _deg(src1, dst1, src2, dst2, do1, di1, do2, di2,
         sidx_v, didx_v, ones_v, stage_v, acc_s, acc_d):
    cid = lax.axis_index("c")
    sid = lax.axis_index("s")

    def setup(i, _):
        ones_v[pl.ds(i * 16, 16)] = jnp.ones((16,), jnp.float32)
        return _
    lax.fori_loop(0, LANE // 16, setup, None)
    _zero_vec_ref(stage_v, ROWS_PER_TILE)
    base = sid * ROWS_PER_TILE
    pltpu.sync_copy(stage_v, acc_s.at[pl.ds(base, ROWS_PER_TILE)])
    pltpu.sync_copy(stage_v, acc_d.at[pl.ds(base, ROWS_PER_TILE)])
    plsc.subcore_barrier()

    def count(src3, dst3, out_s, out_d):
        pltpu.sync_copy(src3.at[sid], sidx_v)
        pltpu.sync_copy(dst3.at[sid], didx_v)

        def body(j, _):
            pltpu.sync_copy(ones_v, acc_s.at[sidx_v.at[j]], add=True)
            pltpu.sync_copy(ones_v, acc_d.at[didx_v.at[j]], add=True)
            return _
        lax.fori_loop(0, ER, body, None)
        plsc.subcore_barrier()
        pltpu.sync_copy(acc_s.at[pl.ds(base, ROWS_PER_TILE)], stage_v)
        pltpu.sync_copy(stage_v, out_s.at[pl.ds(base, ROWS_PER_TILE)])
        pltpu.sync_copy(acc_d.at[pl.ds(base, ROWS_PER_TILE)], stage_v)
        pltpu.sync_copy(stage_v, out_d.at[pl.ds(base, ROWS_PER_TILE)])

    @pl.when(cid == 0)
    def _():
        count(src1, dst1, do1, di1)

    @pl.when(cid == 1)
    def _():
        count(src2, dst2, do2, di2)


@functools.cache
def _agg_kernel():
    return pl.kernel(
        _agg,
        out_type=[jax.ShapeDtypeStruct((NPAD, D), jnp.float32)] * 2,
        mesh=_mesh(),
        scratch_types=[
            pltpu.VMEM((CHK, LANE), jnp.int32),
            pltpu.VMEM((CHK, LANE), jnp.int32),
            pltpu.VMEM((LANE, D), jnp.float32),
            pltpu.VMEM((LANE, D), jnp.float32),
            pltpu.VMEM_SHARED((NPAD, D), jnp.float32),
            pltpu.SemaphoreType.DMA,
            pltpu.SemaphoreType.DMA,
        ],
    )


def _agg(x1, x2, src1, dst1, src2, dst2, out1, out2,
         sidx_v, didx_v, rows0_v, rows1_v, acc, sem0, sem1):
    cid = lax.axis_index("c")
    sid = lax.axis_index("s")

    _zero_rows_ref(rows0_v)
    base = sid * ROWS_PER_TILE
    for chunk in range(ROWS_PER_TILE // LANE):
        pltpu.sync_copy(rows0_v, acc.at[pl.ds(base + chunk * LANE, LANE)])
    plsc.subcore_barrier()

    def run(x_ref, src3, dst3, out_ref):
        def gather_start(j, buf, sem):
            pltpu.async_copy(x_ref.at[sidx_v.at[j]], buf, sem)

        def gather_wait(buf, sem):
            pltpu.make_async_copy(x_ref.at[sidx_v.at[0]], buf, sem).wait()

        def scatter(j, buf):
            pltpu.sync_copy(buf, acc.at[didx_v.at[j]], add=True)

        def chunk_body(c, _):
            pltpu.sync_copy(src3.at[sid, pl.ds(c * CHK, CHK)], sidx_v)
            pltpu.sync_copy(dst3.at[sid, pl.ds(c * CHK, CHK)], didx_v)
            gather_start(0, rows0_v, sem0)

            def body(k, _):
                j0 = 2 * k
                gather_start(j0 + 1, rows1_v, sem1)
                gather_wait(rows0_v, sem0)
                scatter(j0, rows0_v)

                @pl.when(k + 1 < CHK // 2)
                def _():
                    gather_start(j0 + 2, rows0_v, sem0)

                gather_wait(rows1_v, sem1)
                scatter(j0 + 1, rows1_v)
                return _
            lax.fori_loop(0, CHK // 2, body, None)
            return _
        lax.fori_loop(0, ER // CHK, chunk_body, None)
        plsc.subcore_barrier()
        for chunk in range(ROWS_PER_TILE // LANE):
            o = base + chunk * LANE
            pltpu.sync_copy(acc.at[pl.ds(o, LANE)], rows0_v)
            pltpu.sync_copy(rows0_v, out_ref.at[pl.ds(o, LANE)])

    @pl.when(cid == 0)
    def _():
        run(x1, src1, dst1, out1)

    @pl.when(cid == 1)
    def _():
        run(x2, src2, dst2, out2)


def _scale_body(x_ref, d1_ref, d2_ref, o1_ref, o2_ref):
    x = x_ref[...]
    o1_ref[...] = x * lax.rsqrt(jnp.maximum(d1_ref[...], 1.0))
    o2_ref[...] = x * lax.rsqrt(jnp.maximum(d2_ref[...], 1.0))


def _scale(x_pad, do1, do2):
    blk = 1024
    grid = NPAD // blk
    return pl.pallas_call(
        _scale_body,
        grid=(grid,),
        in_specs=[
            pl.BlockSpec((blk, D), lambda i: (i, 0)),
            pl.BlockSpec((blk, 1), lambda i: (i, 0)),
            pl.BlockSpec((blk, 1), lambda i: (i, 0)),
        ],
        out_specs=[
            pl.BlockSpec((blk, D), lambda i: (i, 0)),
            pl.BlockSpec((blk, D), lambda i: (i, 0)),
        ],
        out_shape=[jax.ShapeDtypeStruct((NPAD, D), jnp.float32)] * 2,
    )(x_pad, do1, do2)


def _head_body(a1_ref, a2_ref, d1_ref, d2_ref, w1_ref, b1_ref, w2_ref,
               b2_ref, wo1_ref, wo2_ref, bo_ref, o_ref):
    n1 = lax.rsqrt(jnp.maximum(d1_ref[...], 1.0))
    n2 = lax.rsqrt(jnp.maximum(d2_ref[...], 1.0))
    h1 = jnp.dot(a1_ref[...] * n1, w1_ref[...],
                 preferred_element_type=jnp.float32) + b1_ref[...]
    h2 = jnp.dot(a2_ref[...] * n2, w2_ref[...],
                 preferred_element_type=jnp.float32) + b2_ref[...]
    h1 = jnp.maximum(h1, 0.0)
    h2 = jnp.maximum(h2, 0.0)
    l = (jnp.dot(h1, wo1_ref[...], preferred_element_type=jnp.float32)
         + jnp.dot(h2, wo2_ref[...], preferred_element_type=jnp.float32)
         + bo_ref[...])
    m = jnp.max(l, axis=1, keepdims=True)
    e = jnp.exp(l - m)
    o_ref[...] = l - m - jnp.log(jnp.sum(e, axis=1, keepdims=True))


def _head(agg1, agg2, di1, di2, W1, b1, W2, b2, Wout, bout):
    blk = 1024
    grid = NPAD // blk
    full = lambda shape: pl.BlockSpec(shape, lambda i: tuple(0 for _ in shape))
    return pl.pallas_call(
        _head_body,
        grid=(grid,),
        in_specs=[
            pl.BlockSpec((blk, D), lambda i: (i, 0)),
            pl.BlockSpec((blk, D), lambda i: (i, 0)),
            pl.BlockSpec((blk, 1), lambda i: (i, 0)),
            pl.BlockSpec((blk, 1), lambda i: (i, 0)),
            full((D, H)),
            full((1, H)),
            full((D, H)),
            full((1, H)),
            full((H, C)),
            full((H, C)),
            full((1, C)),
        ],
        out_specs=pl.BlockSpec((blk, C), lambda i: (i, 0)),
        out_shape=jax.ShapeDtypeStruct((NPAD, C), jnp.float32),
    )(agg1, agg2, di1, di2, W1, b1.reshape(1, H), W2, b2.reshape(1, H),
      Wout[:H], Wout[H:], bout.reshape(1, C))


def _pad_edges(g):
    pad = EPAD - E
    src = jnp.concatenate([g[0], jnp.full((pad,), SINK, jnp.int32)])
    dst = jnp.concatenate([g[1], jnp.full((pad,), SINK, jnp.int32)])
    return src.reshape(NS, ER, LANE), dst.reshape(NS, ER, LANE)


def kernel(x, g1, g2, W1, b1, W2, b2, Wout, bout):
    src1, dst1 = _pad_edges(g1)
    src2, dst2 = _pad_edges(g2)
    x_pad = jnp.pad(x, ((0, NPAD - N), (0, 0)))

    do1, di1, do2, di2 = _deg_kernel()(src1, dst1, src2, dst2)
    do1 = do1.reshape(NPAD, 1)
    di1 = di1.reshape(NPAD, 1)
    do2 = do2.reshape(NPAD, 1)
    di2 = di2.reshape(NPAD, 1)

    x1p, x2p = _scale(x_pad, do1, do2)
    agg1, agg2 = _agg_kernel()(x1p, x2p, src1, dst1, src2, dst2)
    out = _head(agg1, agg2, di1, di2, W1, b1, W2, b2, Wout, bout)
    return out[:N]


# speedup vs baseline: 9.4906x; 2.2245x over previous
import functools

import jax
import jax.numpy as jnp
from jax import lax
from jax.experimental import pallas as pl
from jax.experimental.pallas import tpu as pltpu
from jax.experimental.pallas import tpu_sc as plsc

N = 10000
E = 320000
D = 128
H = 128
C = 40

NC = 2
NS = 16
LANE = 128

NPAD = 10240
ROWS_PER_TILE = NPAD // NS
CHK = 32
ER = 160
EPAD = NS * ER * LANE
SINK = N + 16

@functools.cache
def _mesh():
    return plsc.VectorSubcoreMesh(
        core_axis_name="c", subcore_axis_name="s",
        num_cores=NC, num_subcores=NS)


def _zero_vec_ref(ref, n):
    def body(i, _):
        ref[pl.ds(i * 16, 16)] = jnp.zeros((16,), jnp.float32)
        return _
    lax.fori_loop(0, n // 16, body, None)


def _zero_rows_ref(ref):
    def body(k, _):
        i = k // (D // 16)
        l = k % (D // 16)
        ref[i, pl.ds(l * 16, 16)] = jnp.zeros((16,), jnp.float32)
        return _
    lax.fori_loop(0, LANE * (D // 16), body, None)


@functools.cache
def _deg_kernel():
    return pl.kernel(
        _deg,
        out_type=[jax.ShapeDtypeStruct((NPAD,), jnp.float32)] * 4,
        mesh=_mesh(),
        scratch_types=[
            pltpu.VMEM((ER, LANE), jnp.int32),
            pltpu.VMEM((ER, LANE), jnp.int32),
            pltpu.VMEM((LANE,), jnp.float32),
            pltpu.VMEM((ROWS_PER_TILE,), jnp.float32),
            pltpu.VMEM_SHARED((NPAD,), jnp.float32),
            pltpu.VMEM_SHARED((NPAD,), jnp.float32),
        ],
    )


def _deg(src1, dst1, src2, dst2, do1, di1, do2, di2,
         sidx_v, didx_v, ones_v, stage_v, acc_s, acc_d):
    cid = lax.axis_index("c")
    sid = lax.axis_index("s")

    def setup(i, _):
        ones_v[pl.ds(i * 16, 16)] = jnp.ones((16,), jnp.float32)
        return _
    lax.fori_loop(0, LANE // 16, setup, None)
    _zero_vec_ref(stage_v, ROWS_PER_TILE)
    base = sid * ROWS_PER_TILE
    pltpu.sync_copy(stage_v, acc_s.at[pl.ds(base, ROWS_PER_TILE)])
    pltpu.sync_copy(stage_v, acc_d.at[pl.ds(base, ROWS_PER_TILE)])
    plsc.subcore_barrier()

    def count(src3, dst3, out_s, out_d):
        pltpu.sync_copy(src3.at[sid], sidx_v)
        pltpu.sync_copy(dst3.at[sid], didx_v)

        def body(j, _):
            pltpu.sync_copy(ones_v, acc_s.at[sidx_v.at[j]], add=True)
            pltpu.sync_copy(ones_v, acc_d.at[didx_v.at[j]], add=True)
            return _
        lax.fori_loop(0, ER, body, None)
        plsc.subcore_barrier()
        pltpu.sync_copy(acc_s.at[pl.ds(base, ROWS_PER_TILE)], stage_v)
        pltpu.sync_copy(stage_v, out_s.at[pl.ds(base, ROWS_PER_TILE)])
        pltpu.sync_copy(acc_d.at[pl.ds(base, ROWS_PER_TILE)], stage_v)
        pltpu.sync_copy(stage_v, out_d.at[pl.ds(base, ROWS_PER_TILE)])

    @pl.when(cid == 0)
    def _():
        count(src1, dst1, do1, di1)

    @pl.when(cid == 1)
    def _():
        count(src2, dst2, do2, di2)


@functools.cache
def _agg_kernel():
    return pl.kernel(
        _agg,
        out_type=[jax.ShapeDtypeStruct((NPAD, D), jnp.float32)] * 2,
        mesh=_mesh(),
        scratch_types=[
            pltpu.VMEM((CHK, LANE), jnp.int32),
            pltpu.VMEM((CHK, LANE), jnp.int32),
            pltpu.VMEM((LANE, D), jnp.float32),
            pltpu.VMEM((LANE, D), jnp.float32),
            pltpu.VMEM_SHARED((NPAD, D), jnp.float32),
            pltpu.SemaphoreType.DMA,
            pltpu.SemaphoreType.DMA,
        ],
    )


def _agg(x1, x2, src1, dst1, src2, dst2, out1, out2,
         sidx_v, didx_v, rows0_v, rows1_v, acc, sem0, sem1):
    cid = lax.axis_index("c")
    sid = lax.axis_index("s")

    _zero_rows_ref(rows0_v)
    base = sid * ROWS_PER_TILE
    for chunk in range(ROWS_PER_TILE // LANE):
        pltpu.sync_copy(rows0_v, acc.at[pl.ds(base + chunk * LANE, LANE)])
    plsc.subcore_barrier()

    def run(x_ref, src3, dst3, out_ref):
        def gather_start(j, buf, sem):
            pltpu.async_copy(x_ref.at[sidx_v.at[j]], buf, sem)

        def gather_wait(buf, sem):
            pltpu.make_async_copy(x_ref.at[sidx_v.at[0]], buf, sem).wait()

        def scatter(j, buf):
            pltpu.sync_copy(buf, acc.at[didx_v.at[j]], add=True)

        def chunk_body(c, _):
            pltpu.sync_copy(src3.at[sid, pl.ds(c * CHK, CHK)], sidx_v)
            pltpu.sync_copy(dst3.at[sid, pl.ds(c * CHK, CHK)], didx_v)
            gather_start(0, rows0_v, sem0)

            def body(k, _):
                j0 = 2 * k
                gather_start(j0 + 1, rows1_v, sem1)
                gather_wait(rows0_v, sem0)
                scatter(j0, rows0_v)

                @pl.when(k + 1 < CHK // 2)
                def _():
                    gather_start(j0 + 2, rows0_v, sem0)

                gather_wait(rows1_v, sem1)
                scatter(j0 + 1, rows1_v)
                return _
            lax.fori_loop(0, CHK // 2, body, None)
            return _
        lax.fori_loop(0, ER // CHK, chunk_body, None)
        plsc.subcore_barrier()
        for chunk in range(ROWS_PER_TILE // LANE):
            o = base + chunk * LANE
            pltpu.sync_copy(acc.at[pl.ds(o, LANE)], rows0_v)
            pltpu.sync_copy(rows0_v, out_ref.at[pl.ds(o, LANE)])

    @pl.when(cid == 0)
    def _():
        run(x1, src1, dst1, out1)

    @pl.when(cid == 1)
    def _():
        run(x2, src2, dst2, out2)


def _scale_body(x_ref, d1_ref, d2_ref, o1_ref, o2_ref):
    x = x_ref[...]
    o1_ref[...] = x * lax.rsqrt(jnp.maximum(d1_ref[...], 1.0))
    o2_ref[...] = x * lax.rsqrt(jnp.maximum(d2_ref[...], 1.0))


def _scale(x_pad, do1, do2):
    blk = 1024
    grid = NPAD // blk
    return pl.pallas_call(
        _scale_body,
        grid=(grid,),
        in_specs=[
            pl.BlockSpec((blk, D), lambda i: (i, 0)),
            pl.BlockSpec((blk, 1), lambda i: (i, 0)),
            pl.BlockSpec((blk, 1), lambda i: (i, 0)),
        ],
        out_specs=[
            pl.BlockSpec((blk, D), lambda i: (i, 0)),
            pl.BlockSpec((blk, D), lambda i: (i, 0)),
        ],
        out_shape=[jax.ShapeDtypeStruct((NPAD, D), jnp.float32)] * 2,
    )(x_pad, do1, do2)


def _head_body(a1_ref, a2_ref, d1_ref, d2_ref, w1_ref, b1_ref, w2_ref,
               b2_ref, wo1_ref, wo2_ref, bo_ref, o_ref):
    n1 = lax.rsqrt(jnp.maximum(d1_ref[...], 1.0))
    n2 = lax.rsqrt(jnp.maximum(d2_ref[...], 1.0))
    h1 = jnp.dot(a1_ref[...] * n1, w1_ref[...],
                 preferred_element_type=jnp.float32) + b1_ref[...]
    h2 = jnp.dot(a2_ref[...] * n2, w2_ref[...],
                 preferred_element_type=jnp.float32) + b2_ref[...]
    h1 = jnp.maximum(h1, 0.0)
    h2 = jnp.maximum(h2, 0.0)
    l = (jnp.dot(h1, wo1_ref[...], preferred_element_type=jnp.float32)
         + jnp.dot(h2, wo2_ref[...], preferred_element_type=jnp.float32)
         + bo_ref[...])
    m = jnp.max(l, axis=1, keepdims=True)
    e = jnp.exp(l - m)
    o_ref[...] = l - m - jnp.log(jnp.sum(e, axis=1, keepdims=True))


def _head(agg1, agg2, di1, di2, W1, b1, W2, b2, Wout, bout):
    blk = 1024
    grid = NPAD // blk
    full = lambda shape: pl.BlockSpec(shape, lambda i: tuple(0 for _ in shape))
    return pl.pallas_call(
        _head_body,
        grid=(grid,),
        in_specs=[
            pl.BlockSpec((blk, D), lambda i: (i, 0)),
            pl.BlockSpec((blk, D), lambda i: (i, 0)),
            pl.BlockSpec((blk, 1), lambda i: (i, 0)),
            pl.BlockSpec((blk, 1), lambda i: (i, 0)),
            full((D, H)),
            full((1, H)),
            full((D, H)),
            full((1, H)),
            full((H, C)),
            full((H, C)),
            full((1, C)),
        ],
        out_specs=pl.BlockSpec((blk, C), lambda i: (i, 0)),
        out_shape=jax.ShapeDtypeStruct((NPAD, C), jnp.float32),
    )(agg1, agg2, di1, di2, W1, b1.reshape(1, H), W2, b2.reshape(1, H),
      Wout[:H], Wout[H:], bout.reshape(1, C))


def _pad_edges(g):
    pad = EPAD - E
    fill = N + jnp.arange(pad, dtype=jnp.int32) % (NPAD - N)
    src = jnp.concatenate([g[0], fill])
    dst = jnp.concatenate([g[1], fill])
    return src.reshape(NS, ER, LANE), dst.reshape(NS, ER, LANE)


def kernel(x, g1, g2, W1, b1, W2, b2, Wout, bout):
    src1, dst1 = _pad_edges(g1)
    src2, dst2 = _pad_edges(g2)
    x_pad = jnp.pad(x, ((0, NPAD - N), (0, 0)))

    do1, di1, do2, di2 = _deg_kernel()(src1, dst1, src2, dst2)
    do1 = do1.reshape(NPAD, 1)
    di1 = di1.reshape(NPAD, 1)
    do2 = do2.reshape(NPAD, 1)
    di2 = di2.reshape(NPAD, 1)

    x1p, x2p = _scale(x_pad, do1, do2)
    agg1, agg2 = _agg_kernel()(x1p, x2p, src1, dst1, src2, dst2)
    out = _head(agg1, agg2, di1, di2, W1, b1, W2, b2, Wout, bout)
    return out[:N]


# speedup vs baseline: 9.5100x; 1.0020x over previous
import functools

import numpy as np

import jax
import jax.numpy as jnp
from jax import lax
from jax.experimental import pallas as pl
from jax.experimental.pallas import tpu as pltpu
from jax.experimental.pallas import tpu_sc as plsc

N = 10000
E = 320000
D = 128
H = 128
C = 40

NC = 2
NS = 16
LANE = 128

NPAD = 10240
ROWS_PER_TILE = NPAD // NS
NBUF = 2
CHK = 32
ER = 160
EPAD = NS * ER * LANE
SINK = N + 16

@functools.cache
def _mesh():
    return plsc.VectorSubcoreMesh(
        core_axis_name="c", subcore_axis_name="s",
        num_cores=NC, num_subcores=NS)


def _zero_vec_ref(ref, n):
    def body(i, _):
        ref[pl.ds(i * 16, 16)] = jnp.zeros((16,), jnp.float32)
        return _
    lax.fori_loop(0, n // 16, body, None)


def _zero_rows_ref(ref):
    def body(k, _):
        i = k // (D // 16)
        l = k % (D // 16)
        ref[i, pl.ds(l * 16, 16)] = jnp.zeros((16,), jnp.float32)
        return _
    lax.fori_loop(0, LANE * (D // 16), body, None)


@functools.cache
def _deg_kernel():
    return pl.kernel(
        _deg,
        out_type=[jax.ShapeDtypeStruct((NPAD,), jnp.float32)] * 4,
        mesh=_mesh(),
        scratch_types=[
            pltpu.VMEM((ER, LANE), jnp.int32),
            pltpu.VMEM((ER, LANE), jnp.int32),
            pltpu.VMEM((LANE,), jnp.float32),
            pltpu.VMEM((ROWS_PER_TILE,), jnp.float32),
            pltpu.VMEM_SHARED((NPAD,), jnp.float32),
            pltpu.VMEM_SHARED((NPAD,), jnp.float32),
        ],
    )


def _deg(src1, dst1, src2, dst2, do1, di1, do2, di2,
         sidx_v, didx_v, ones_v, stage_v, acc_s, acc_d):
    cid = lax.axis_index("c")
    sid = lax.axis_index("s")

    def setup(i, _):
        ones_v[pl.ds(i * 16, 16)] = jnp.ones((16,), jnp.float32)
        return _
    lax.fori_loop(0, LANE // 16, setup, None)
    _zero_vec_ref(stage_v, ROWS_PER_TILE)
    base = sid * ROWS_PER_TILE
    pltpu.sync_copy(stage_v, acc_s.at[pl.ds(base, ROWS_PER_TILE)])
    pltpu.sync_copy(stage_v, acc_d.at[pl.ds(base, ROWS_PER_TILE)])
    plsc.subcore_barrier()

    def count(src3, dst3, out_s, out_d):
        pltpu.sync_copy(src3.at[sid], sidx_v)
        pltpu.sync_copy(dst3.at[sid], didx_v)

        def body(j, _):
            pltpu.sync_copy(ones_v, acc_s.at[sidx_v.at[j]], add=True)
            pltpu.sync_copy(ones_v, acc_d.at[didx_v.at[j]], add=True)
            return _
        lax.fori_loop(0, ER, body, None)
        plsc.subcore_barrier()
        pltpu.sync_copy(acc_s.at[pl.ds(base, ROWS_PER_TILE)], stage_v)
        pltpu.sync_copy(stage_v, out_s.at[pl.ds(base, ROWS_PER_TILE)])
        pltpu.sync_copy(acc_d.at[pl.ds(base, ROWS_PER_TILE)], stage_v)
        pltpu.sync_copy(stage_v, out_d.at[pl.ds(base, ROWS_PER_TILE)])

    @pl.when(cid == 0)
    def _():
        count(src1, dst1, do1, di1)

    @pl.when(cid == 1)
    def _():
        count(src2, dst2, do2, di2)


@functools.cache
def _agg_kernel():
    return pl.kernel(
        _agg,
        out_type=[jax.ShapeDtypeStruct((NPAD, D), jnp.float32)] * 2,
        mesh=_mesh(),
        scratch_types=[
            pltpu.VMEM((CHK, LANE), jnp.int32),
            pltpu.VMEM((CHK, LANE), jnp.int32),
            pltpu.VMEM((NBUF, LANE, D), jnp.float32),
            pltpu.VMEM_SHARED((NPAD, D), jnp.float32),
        ] + [pltpu.SemaphoreType.DMA] * NBUF,
    )


def _agg(x1, x2, src1, dst1, src2, dst2, out1, out2,
         sidx_v, didx_v, rows_v, acc, *sems):
    cid = lax.axis_index("c")
    sid = lax.axis_index("s")

    _zero_rows_ref(rows_v.at[0])
    base = sid * ROWS_PER_TILE
    for chunk in range(ROWS_PER_TILE // LANE):
        pltpu.sync_copy(rows_v.at[0], acc.at[pl.ds(base + chunk * LANE, LANE)])
    plsc.subcore_barrier()

    def run(x_ref, src3, dst3, out_ref):
        def gather_start(j, b):
            pltpu.async_copy(x_ref.at[sidx_v.at[j]], rows_v.at[b], sems[b])

        def gather_wait(b):
            pltpu.make_async_copy(
                x_ref.at[sidx_v.at[0]], rows_v.at[b], sems[b]).wait()

        def scatter(j, b):
            pltpu.sync_copy(rows_v.at[b], acc.at[didx_v.at[j]], add=True)

        def chunk_body(c, _):
            pltpu.sync_copy(src3.at[sid, pl.ds(c * CHK, CHK)], sidx_v)
            pltpu.sync_copy(dst3.at[sid, pl.ds(c * CHK, CHK)], didx_v)
            for b in range(NBUF - 1):
                gather_start(b, b)

            def body(k, _):
                j0 = k * NBUF
                for b in range(NBUF):
                    nxt = j0 + b + NBUF - 1

                    @pl.when(nxt < CHK)
                    def _():
                        gather_start(nxt, (b + NBUF - 1) % NBUF)

                    gather_wait(b)
                    scatter(j0 + b, b)
                return _
            lax.fori_loop(0, CHK // NBUF, body, None)
            return _
        lax.fori_loop(0, ER // CHK, chunk_body, None)
        plsc.subcore_barrier()
        for chunk in range(ROWS_PER_TILE // LANE):
            o = base + chunk * LANE
            pltpu.sync_copy(acc.at[pl.ds(o, LANE)], rows_v.at[0])
            pltpu.sync_copy(rows_v.at[0], out_ref.at[pl.ds(o, LANE)])

    @pl.when(cid == 0)
    def _():
        run(x1, src1, dst1, out1)

    @pl.when(cid == 1)
    def _():
        run(x2, src2, dst2, out2)


def _scale_body(x_ref, d1_ref, d2_ref, o1_ref, o2_ref):
    x = x_ref[...]
    o1_ref[...] = x * lax.rsqrt(jnp.maximum(d1_ref[...], 1.0))
    o2_ref[...] = x * lax.rsqrt(jnp.maximum(d2_ref[...], 1.0))


def _scale(x_pad, do1, do2):
    blk = 1024
    grid = NPAD // blk
    return pl.pallas_call(
        _scale_body,
        grid=(grid,),
        in_specs=[
            pl.BlockSpec((blk, D), lambda i: (i, 0)),
            pl.BlockSpec((blk, 1), lambda i: (i, 0)),
            pl.BlockSpec((blk, 1), lambda i: (i, 0)),
        ],
        out_specs=[
            pl.BlockSpec((blk, D), lambda i: (i, 0)),
            pl.BlockSpec((blk, D), lambda i: (i, 0)),
        ],
        out_shape=[jax.ShapeDtypeStruct((NPAD, D), jnp.float32)] * 2,
    )(x_pad, do1, do2)


def _head_body(a1_ref, a2_ref, d1_ref, d2_ref, w1_ref, b1_ref, w2_ref,
               b2_ref, wo1_ref, wo2_ref, bo_ref, o_ref):
    n1 = lax.rsqrt(jnp.maximum(d1_ref[...], 1.0))
    n2 = lax.rsqrt(jnp.maximum(d2_ref[...], 1.0))
    h1 = jnp.dot(a1_ref[...] * n1, w1_ref[...],
                 preferred_element_type=jnp.float32) + b1_ref[...]
    h2 = jnp.dot(a2_ref[...] * n2, w2_ref[...],
                 preferred_element_type=jnp.float32) + b2_ref[...]
    h1 = jnp.maximum(h1, 0.0)
    h2 = jnp.maximum(h2, 0.0)
    l = (jnp.dot(h1, wo1_ref[...], preferred_element_type=jnp.float32)
         + jnp.dot(h2, wo2_ref[...], preferred_element_type=jnp.float32)
         + bo_ref[...])
    m = jnp.max(l, axis=1, keepdims=True)
    e = jnp.exp(l - m)
    o_ref[...] = l - m - jnp.log(jnp.sum(e, axis=1, keepdims=True))


def _head(agg1, agg2, di1, di2, W1, b1, W2, b2, Wout, bout):
    blk = 1000
    grid = N // blk
    full = lambda shape: pl.BlockSpec(shape, lambda i: tuple(0 for _ in shape))
    return pl.pallas_call(
        _head_body,
        grid=(grid,),
        in_specs=[
            pl.BlockSpec((blk, D), lambda i: (i, 0)),
            pl.BlockSpec((blk, D), lambda i: (i, 0)),
            pl.BlockSpec((blk, 1), lambda i: (i, 0)),
            pl.BlockSpec((blk, 1), lambda i: (i, 0)),
            full((D, H)),
            full((1, H)),
            full((D, H)),
            full((1, H)),
            full((H, C)),
            full((H, C)),
            full((1, C)),
        ],
        out_specs=pl.BlockSpec((blk, C), lambda i: (i, 0)),
        out_shape=jax.ShapeDtypeStruct((N, C), jnp.float32),
    )(agg1, agg2, di1, di2, W1, b1.reshape(1, H), W2, b2.reshape(1, H),
      Wout[:H], Wout[H:], bout.reshape(1, C))


_PAD_FILL = np.asarray(
    N + np.arange(EPAD - E, dtype=np.int32) % (NPAD - N), np.int32)


def _pad_edges(g):
    src = jnp.concatenate([g[0], jnp.asarray(_PAD_FILL)])
    dst = jnp.concatenate([g[1], jnp.asarray(_PAD_FILL)])
    return src.reshape(NS, ER, LANE), dst.reshape(NS, ER, LANE)


def kernel(x, g1, g2, W1, b1, W2, b2, Wout, bout):
    src1, dst1 = _pad_edges(g1)
    src2, dst2 = _pad_edges(g2)
    x_pad = jnp.pad(x, ((0, NPAD - N), (0, 0)))

    do1, di1, do2, di2 = _deg_kernel()(src1, dst1, src2, dst2)
    do1 = do1.reshape(NPAD, 1)
    di1 = di1.reshape(NPAD, 1)
    do2 = do2.reshape(NPAD, 1)
    di2 = di2.reshape(NPAD, 1)

    x1p, x2p = _scale(x_pad, do1, do2)
    agg1, agg2 = _agg_kernel()(x1p, x2p, src1, dst1, src2, dst2)
    return _head(agg1, agg2, di1, di2, W1, b1, W2, b2, Wout, bout)


# speedup vs baseline: 10.1817x; 1.0706x over previous
import functools

import numpy as np

import jax
import jax.numpy as jnp
from jax import lax
from jax.experimental import pallas as pl
from jax.experimental.pallas import tpu as pltpu
from jax.experimental.pallas import tpu_sc as plsc

N = 10000
E = 320000
D = 128
H = 128
C = 40

NC = 2
NS = 16
LANE = 128

NPAD = 10240
ROWS_PER_TILE = NPAD // NS
NBUF = 2
CHK = 32
ER = 160
EPAD = NS * ER * LANE
SINK = N + 16

@functools.cache
def _mesh():
    return plsc.VectorSubcoreMesh(
        core_axis_name="c", subcore_axis_name="s",
        num_cores=NC, num_subcores=NS)


def _zero_vec_ref(ref, n):
    def body(i, _):
        ref[pl.ds(i * 16, 16)] = jnp.zeros((16,), jnp.float32)
        return _
    lax.fori_loop(0, n // 16, body, None)


def _zero_rows_ref(ref):
    def body(k, _):
        i = k // (D // 16)
        l = k % (D // 16)
        ref[i, pl.ds(l * 16, 16)] = jnp.zeros((16,), jnp.float32)
        return _
    lax.fori_loop(0, LANE * (D // 16), body, None)


@functools.cache
def _deg_kernel():
    return pl.kernel(
        _deg,
        out_type=[jax.ShapeDtypeStruct((NPAD,), jnp.float32)] * 4,
        mesh=_mesh(),
        scratch_types=[
            pltpu.VMEM((ER, LANE), jnp.int32),
            pltpu.VMEM((ER, LANE), jnp.int32),
            pltpu.VMEM((LANE,), jnp.float32),
            pltpu.VMEM((ROWS_PER_TILE,), jnp.float32),
            pltpu.VMEM_SHARED((NPAD,), jnp.float32),
            pltpu.VMEM_SHARED((NPAD,), jnp.float32),
            pltpu.SemaphoreType.DMA,
        ],
    )


def _deg(src1, dst1, src2, dst2, do1, di1, do2, di2,
         sidx_v, didx_v, ones_v, stage_v, acc_s, acc_d, sem):
    cid = lax.axis_index("c")
    sid = lax.axis_index("s")

    def setup(i, _):
        ones_v[pl.ds(i * 16, 16)] = jnp.ones((16,), jnp.float32)
        return _
    lax.fori_loop(0, LANE // 16, setup, None)
    _zero_vec_ref(stage_v, ROWS_PER_TILE)
    base = sid * ROWS_PER_TILE
    pltpu.sync_copy(stage_v, acc_s.at[pl.ds(base, ROWS_PER_TILE)])
    pltpu.sync_copy(stage_v, acc_d.at[pl.ds(base, ROWS_PER_TILE)])
    plsc.subcore_barrier()

    def count(src3, dst3, out_s, out_d):
        pltpu.sync_copy(src3.at[sid], sidx_v)
        pltpu.sync_copy(dst3.at[sid], didx_v)

        def body(j, _):
            pltpu.async_copy(ones_v, acc_s.at[sidx_v.at[j]], sem, add=True)
            pltpu.async_copy(ones_v, acc_d.at[didx_v.at[j]], sem, add=True)
            return _
        lax.fori_loop(0, ER, body, None)

        def drain(j, _):
            pltpu.make_async_copy(
                ones_v, acc_s.at[sidx_v.at[0]], sem).wait()
            pltpu.make_async_copy(
                ones_v, acc_d.at[didx_v.at[0]], sem).wait()
            return _
        lax.fori_loop(0, ER, drain, None)
        plsc.subcore_barrier()
        pltpu.sync_copy(acc_s.at[pl.ds(base, ROWS_PER_TILE)], stage_v)
        pltpu.sync_copy(stage_v, out_s.at[pl.ds(base, ROWS_PER_TILE)])
        pltpu.sync_copy(acc_d.at[pl.ds(base, ROWS_PER_TILE)], stage_v)
        pltpu.sync_copy(stage_v, out_d.at[pl.ds(base, ROWS_PER_TILE)])

    @pl.when(cid == 0)
    def _():
        count(src1, dst1, do1, di1)

    @pl.when(cid == 1)
    def _():
        count(src2, dst2, do2, di2)


@functools.cache
def _agg_kernel():
    return pl.kernel(
        _agg,
        out_type=[jax.ShapeDtypeStruct((NPAD, D), jnp.float32)] * 2,
        mesh=_mesh(),
        scratch_types=[
            pltpu.VMEM((CHK, LANE), jnp.int32),
            pltpu.VMEM((CHK, LANE), jnp.int32),
            pltpu.VMEM((NBUF, LANE, D), jnp.float32),
            pltpu.VMEM_SHARED((NPAD, D), jnp.float32),
        ] + [pltpu.SemaphoreType.DMA] * NBUF,
    )


def _agg(x1, x2, src1, dst1, src2, dst2, out1, out2,
         sidx_v, didx_v, rows_v, acc, *sems):
    cid = lax.axis_index("c")
    sid = lax.axis_index("s")

    _zero_rows_ref(rows_v.at[0])
    base = sid * ROWS_PER_TILE
    for chunk in range(ROWS_PER_TILE // LANE):
        pltpu.sync_copy(rows_v.at[0], acc.at[pl.ds(base + chunk * LANE, LANE)])
    plsc.subcore_barrier()

    def run(x_ref, src3, dst3, out_ref):
        def gather_start(j, b):
            pltpu.async_copy(x_ref.at[sidx_v.at[j]], rows_v.at[b], sems[b])

        def gather_wait(b):
            pltpu.make_async_copy(
                x_ref.at[sidx_v.at[0]], rows_v.at[b], sems[b]).wait()

        def scatter(j, b):
            pltpu.sync_copy(rows_v.at[b], acc.at[didx_v.at[j]], add=True)

        def chunk_body(c, _):
            pltpu.sync_copy(src3.at[sid, pl.ds(c * CHK, CHK)], sidx_v)
            pltpu.sync_copy(dst3.at[sid, pl.ds(c * CHK, CHK)], didx_v)
            for b in range(NBUF - 1):
                gather_start(b, b)

            def body(k, _):
                j0 = k * NBUF
                for b in range(NBUF):
                    nxt = j0 + b + NBUF - 1

                    @pl.when(nxt < CHK)
                    def _():
                        gather_start(nxt, (b + NBUF - 1) % NBUF)

                    gather_wait(b)
                    scatter(j0 + b, b)
                return _
            lax.fori_loop(0, CHK // NBUF, body, None)
            return _
        lax.fori_loop(0, ER // CHK, chunk_body, None)
        plsc.subcore_barrier()
        for chunk in range(ROWS_PER_TILE // LANE):
            o = base + chunk * LANE
            pltpu.sync_copy(acc.at[pl.ds(o, LANE)], rows_v.at[0])
            pltpu.sync_copy(rows_v.at[0], out_ref.at[pl.ds(o, LANE)])

    @pl.when(cid == 0)
    def _():
        run(x1, src1, dst1, out1)

    @pl.when(cid == 1)
    def _():
        run(x2, src2, dst2, out2)


def _scale_body(x_ref, d1_ref, d2_ref, o1_ref, o2_ref):
    x = x_ref[...]
    o1_ref[...] = x * lax.rsqrt(jnp.maximum(d1_ref[...], 1.0))
    o2_ref[...] = x * lax.rsqrt(jnp.maximum(d2_ref[...], 1.0))


def _scale(x_pad, do1, do2):
    blk = 1024
    grid = NPAD // blk
    return pl.pallas_call(
        _scale_body,
        grid=(grid,),
        in_specs=[
            pl.BlockSpec((blk, D), lambda i: (i, 0)),
            pl.BlockSpec((blk, 1), lambda i: (i, 0)),
            pl.BlockSpec((blk, 1), lambda i: (i, 0)),
        ],
        out_specs=[
            pl.BlockSpec((blk, D), lambda i: (i, 0)),
            pl.BlockSpec((blk, D), lambda i: (i, 0)),
        ],
        out_shape=[jax.ShapeDtypeStruct((NPAD, D), jnp.float32)] * 2,
    )(x_pad, do1, do2)


def _head_body(a1_ref, a2_ref, d1_ref, d2_ref, w1_ref, b1_ref, w2_ref,
               b2_ref, wo1_ref, wo2_ref, bo_ref, o_ref):
    n1 = lax.rsqrt(jnp.maximum(d1_ref[...], 1.0))
    n2 = lax.rsqrt(jnp.maximum(d2_ref[...], 1.0))
    h1 = jnp.dot(a1_ref[...] * n1, w1_ref[...],
                 preferred_element_type=jnp.float32) + b1_ref[...]
    h2 = jnp.dot(a2_ref[...] * n2, w2_ref[...],
                 preferred_element_type=jnp.float32) + b2_ref[...]
    h1 = jnp.maximum(h1, 0.0)
    h2 = jnp.maximum(h2, 0.0)
    l = (jnp.dot(h1, wo1_ref[...], preferred_element_type=jnp.float32)
         + jnp.dot(h2, wo2_ref[...], preferred_element_type=jnp.float32)
         + bo_ref[...])
    m = jnp.max(l, axis=1, keepdims=True)
    e = jnp.exp(l - m)
    o_ref[...] = l - m - jnp.log(jnp.sum(e, axis=1, keepdims=True))


def _head(agg1, agg2, di1, di2, W1, b1, W2, b2, Wout, bout):
    blk = 1000
    grid = N // blk
    full = lambda shape: pl.BlockSpec(shape, lambda i: tuple(0 for _ in shape))
    return pl.pallas_call(
        _head_body,
        grid=(grid,),
        in_specs=[
            pl.BlockSpec((blk, D), lambda i: (i, 0)),
            pl.BlockSpec((blk, D), lambda i: (i, 0)),
            pl.BlockSpec((blk, 1), lambda i: (i, 0)),
            pl.BlockSpec((blk, 1), lambda i: (i, 0)),
            full((D, H)),
            full((1, H)),
            full((D, H)),
            full((1, H)),
            full((H, C)),
            full((H, C)),
            full((1, C)),
        ],
        out_specs=pl.BlockSpec((blk, C), lambda i: (i, 0)),
        out_shape=jax.ShapeDtypeStruct((N, C), jnp.float32),
    )(agg1, agg2, di1, di2, W1, b1.reshape(1, H), W2, b2.reshape(1, H),
      Wout[:H], Wout[H:], bout.reshape(1, C))


_PAD_FILL = np.asarray(
    N + np.arange(EPAD - E, dtype=np.int32) % (NPAD - N), np.int32)


def _pad_edges(g):
    src = jnp.concatenate([g[0], jnp.asarray(_PAD_FILL)])
    dst = jnp.concatenate([g[1], jnp.asarray(_PAD_FILL)])
    return src.reshape(NS, ER, LANE), dst.reshape(NS, ER, LANE)


def kernel(x, g1, g2, W1, b1, W2, b2, Wout, bout):
    src1, dst1 = _pad_edges(g1)
    src2, dst2 = _pad_edges(g2)
    x_pad = jnp.pad(x, ((0, NPAD - N), (0, 0)))

    do1, di1, do2, di2 = _deg_kernel()(src1, dst1, src2, dst2)
    do1 = do1.reshape(NPAD, 1)
    di1 = di1.reshape(NPAD, 1)
    do2 = do2.reshape(NPAD, 1)
    di2 = di2.reshape(NPAD, 1)

    x1p, x2p = _scale(x_pad, do1, do2)
    agg1, agg2 = _agg_kernel()(x1p, x2p, src1, dst1, src2, dst2)
    return _head(agg1, agg2, di1, di2, W1, b1, W2, b2, Wout, bout)
